# baseline (speedup 1.0000x reference)
import functools

import jax
import jax.numpy as jnp
from jax.experimental import pallas as pl

K = 2
E = 8
M0, M1 = 64, 64
N0, N1 = 64, 64
D_IN = M0 * M1
D_OUT = N0 * N1

BT = 256


def _body(x_ref, gw_ref, w1_ref, w2_ref, b_ref, o_ref):
    xb2 = x_ref[...].astype(jnp.bfloat16)
    xb = xb2.reshape(BT, M1, M0)
    logits = jax.lax.dot_general(
        gw_ref[...], xb2, (((1,), (1,)), ((), ())),
        preferred_element_type=jnp.float32)
    eidx = jax.lax.broadcasted_iota(jnp.int32, (E, BT), 0)
    m1 = jnp.max(logits, axis=0, keepdims=True)
    i1 = jnp.argmax(logits, axis=0)
    mask1 = eidx == i1[None, :]
    neg = jnp.finfo(jnp.float32).min
    logits2 = jnp.where(mask1, neg, logits)
    m2 = jnp.max(logits2, axis=0, keepdims=True)
    i2 = jnp.argmax(logits2, axis=0)
    mask2 = eidx == i2[None, :]
    z = jnp.exp(m2 - m1)
    w1v = 1.0 / (1.0 + z) + 1e-6
    w2v = z / (1.0 + z) + 1e-6
    sw = jnp.where(mask1, w1v, 0.0) + jnp.where(mask2, w2v, 0.0)

    t2 = jax.lax.dot_general(
        w1_ref[...], xb, (((1,), (2,)), ((0,), (1,))),
        preferred_element_type=jnp.float32)
    g = jnp.broadcast_to(sw[None, :, :], (N0, E, BT)).reshape(N0 * E, BT)
    t2 = (t2 * g[None, :, :]).astype(jnp.bfloat16)
    w2 = w2_ref[...]
    for k in range(N0):
        t2k = t2[:, k * E:(k + 1) * E, :].reshape(M1 * E, BT)
        yk = jax.lax.dot_general(
            t2k, w2[k], (((0,), (0,)), ((), ())),
            preferred_element_type=jnp.float32)
        o_ref[:, k * N1:(k + 1) * N1] = yk + b_ref[:, k * N1:(k + 1) * N1]


@functools.partial(jax.jit, static_argnames=("interpret",))
def _run(xr, gw, W1, W2, b, interpret=False):
    T = xr.shape[0]
    grid = (T // BT,)
    return pl.pallas_call(
        _body,
        grid=grid,
        in_specs=[
            pl.BlockSpec((BT, D_IN), lambda i: (i, 0)),
            pl.BlockSpec((E, D_IN), lambda i: (0, 0)),
            pl.BlockSpec((M1, M0, N0 * E), lambda i: (0, 0, 0)),
            pl.BlockSpec((N0, M1 * E, N1), lambda i: (0, 0, 0)),
            pl.BlockSpec((1, D_OUT), lambda i: (0, 0)),
        ],
        out_specs=pl.BlockSpec((BT, D_OUT), lambda i: (i, 0)),
        out_shape=jax.ShapeDtypeStruct((T, D_OUT), jnp.float32),
        interpret=interpret,
    )(xr, gw, W1, W2, b)


def kernel(inputs, gate_W, W1, W2, b):
    batch_shape = inputs.shape[:-1]
    xr = inputs.reshape(-1, inputs.shape[-1])
    gw = gate_W.astype(jnp.bfloat16)
    out = _run(xr, gw, W1.reshape(M1, M0, N0 * E).astype(jnp.bfloat16),
               W2.reshape(N0, M1 * E, N1).astype(jnp.bfloat16),
               b.reshape(1, D_OUT))
    return out.reshape(*batch_shape, D_OUT)


# speedup vs baseline: 1.0572x; 1.0572x over previous
import functools

import jax
import jax.numpy as jnp
from jax.experimental import pallas as pl

K = 2
E = 8
M0, M1 = 64, 64
N0, N1 = 64, 64
D_IN = M0 * M1
D_OUT = N0 * N1

BT = 256


def _body(x_ref, gw_ref, w1_ref, w2_ref, b_ref, o_ref):
    xb2 = x_ref[...].astype(jnp.bfloat16)
    xb = xb2.reshape(BT, M1, M0)
    logits = jax.lax.dot_general(
        gw_ref[...], xb2, (((1,), (1,)), ((), ())),
        preferred_element_type=jnp.float32)
    eidx = jax.lax.broadcasted_iota(jnp.int32, (E, BT), 0)
    m1 = jnp.max(logits, axis=0, keepdims=True)
    i1 = jnp.argmax(logits, axis=0)
    mask1 = eidx == i1[None, :]
    neg = jnp.finfo(jnp.float32).min
    logits2 = jnp.where(mask1, neg, logits)
    m2 = jnp.max(logits2, axis=0, keepdims=True)
    i2 = jnp.argmax(logits2, axis=0)
    mask2 = eidx == i2[None, :]
    z = jnp.exp(m2 - m1)
    w1v = 1.0 / (1.0 + z) + 1e-6
    w2v = z / (1.0 + z) + 1e-6
    sw = jnp.where(mask1, w1v, 0.0) + jnp.where(mask2, w2v, 0.0)

    t2 = jax.lax.dot_general(
        w1_ref[...], xb, (((1,), (2,)), ((0,), (1,))),
        preferred_element_type=jnp.float32)
    g = jnp.broadcast_to(sw[None, :, :], (N0, E, BT)).reshape(N0 * E, BT)
    t2 = (t2 * g[None, :, :]).astype(jnp.bfloat16)
    w2 = w2_ref[...]
    ys = []
    for k in range(N0):
        t2k = t2[:, k * E:(k + 1) * E, :].reshape(M1 * E, BT)
        ys.append(jax.lax.dot_general(
            t2k, w2[k], (((0,), (0,)), ((), ())),
            preferred_element_type=jnp.float32))
    y = jnp.concatenate(ys, axis=1)
    o_ref[...] = y + b_ref[...]


@functools.partial(jax.jit, static_argnames=("interpret",))
def _run(xr, gw, W1, W2, b, interpret=False):
    T = xr.shape[0]
    grid = (T // BT,)
    return pl.pallas_call(
        _body,
        grid=grid,
        in_specs=[
            pl.BlockSpec((BT, D_IN), lambda i: (i, 0)),
            pl.BlockSpec((E, D_IN), lambda i: (0, 0)),
            pl.BlockSpec((M1, M0, N0 * E), lambda i: (0, 0, 0)),
            pl.BlockSpec((N0, M1 * E, N1), lambda i: (0, 0, 0)),
            pl.BlockSpec((1, D_OUT), lambda i: (0, 0)),
        ],
        out_specs=pl.BlockSpec((BT, D_OUT), lambda i: (i, 0)),
        out_shape=jax.ShapeDtypeStruct((T, D_OUT), jnp.float32),
        interpret=interpret,
    )(xr, gw, W1, W2, b)


def kernel(inputs, gate_W, W1, W2, b):
    batch_shape = inputs.shape[:-1]
    xr = inputs.reshape(-1, inputs.shape[-1])
    gw = gate_W.astype(jnp.bfloat16)
    out = _run(xr, gw, W1.reshape(M1, M0, N0 * E).astype(jnp.bfloat16),
               W2.reshape(N0, M1 * E, N1).astype(jnp.bfloat16),
               b.reshape(1, D_OUT))
    return out.reshape(*batch_shape, D_OUT)


# speedup vs baseline: 1.0599x; 1.0026x over previous
import functools

import jax
import jax.numpy as jnp
from jax.experimental import pallas as pl
from jax.experimental.pallas import tpu as pltpu

K = 2
E = 8
M0, M1 = 64, 64
N0, N1 = 64, 64
D_IN = M0 * M1
D_OUT = N0 * N1

BT = 256


def _body(x_ref, gw_ref, w1_ref, w2_ref, b_ref, o_ref):
    xb2 = x_ref[...].astype(jnp.bfloat16)
    xb = xb2.reshape(BT, M1, M0)
    logits = jax.lax.dot_general(
        gw_ref[...], xb2, (((1,), (1,)), ((), ())),
        preferred_element_type=jnp.float32)
    eidx = jax.lax.broadcasted_iota(jnp.int32, (E, BT), 0)
    m1 = jnp.max(logits, axis=0, keepdims=True)
    i1 = jnp.argmax(logits, axis=0)
    mask1 = eidx == i1[None, :]
    neg = jnp.finfo(jnp.float32).min
    logits2 = jnp.where(mask1, neg, logits)
    m2 = jnp.max(logits2, axis=0, keepdims=True)
    i2 = jnp.argmax(logits2, axis=0)
    mask2 = eidx == i2[None, :]
    z = jnp.exp(m2 - m1)
    w1v = 1.0 / (1.0 + z) + 1e-6
    w2v = z / (1.0 + z) + 1e-6
    sw = jnp.where(mask1, w1v, 0.0) + jnp.where(mask2, w2v, 0.0)

    t2 = jax.lax.dot_general(
        w1_ref[...], xb, (((1,), (2,)), ((0,), (1,))),
        preferred_element_type=jnp.float32)
    g = jnp.broadcast_to(sw[None, :, :], (N0, E, BT)).reshape(N0 * E, BT)
    t2 = (t2 * g[None, :, :]).astype(jnp.bfloat16)
    w2 = w2_ref[...]
    ys = []
    for k in range(N0):
        t2k = t2[:, k * E:(k + 1) * E, :].reshape(M1 * E, BT)
        ys.append(jax.lax.dot_general(
            t2k, w2[k], (((0,), (0,)), ((), ())),
            preferred_element_type=jnp.float32))
    y = jnp.concatenate(ys, axis=1)
    o_ref[...] = y + b_ref[...]


@functools.partial(jax.jit, static_argnames=("interpret",))
def _run(xr, gw, W1, W2, b, interpret=False):
    T = xr.shape[0]
    grid = (T // BT,)
    return pl.pallas_call(
        _body,
        grid=grid,
        in_specs=[
            pl.BlockSpec((BT, D_IN), lambda i: (i, 0)),
            pl.BlockSpec((E, D_IN), lambda i: (0, 0)),
            pl.BlockSpec((M1, M0, N0 * E), lambda i: (0, 0, 0)),
            pl.BlockSpec((N0, M1 * E, N1), lambda i: (0, 0, 0)),
            pl.BlockSpec((1, D_OUT), lambda i: (0, 0)),
        ],
        out_specs=pl.BlockSpec((BT, D_OUT), lambda i: (i, 0)),
        out_shape=jax.ShapeDtypeStruct((T, D_OUT), jnp.float32),
        compiler_params=pltpu.CompilerParams(
            dimension_semantics=("parallel",)),
        interpret=interpret,
    )(xr, gw, W1, W2, b)


def kernel(inputs, gate_W, W1, W2, b):
    batch_shape = inputs.shape[:-1]
    xr = inputs.reshape(-1, inputs.shape[-1])
    gw = gate_W.astype(jnp.bfloat16)
    out = _run(xr, gw, W1.reshape(M1, M0, N0 * E).astype(jnp.bfloat16),
               W2.reshape(N0, M1 * E, N1).astype(jnp.bfloat16),
               b.reshape(1, D_OUT))
    return out.reshape(*batch_shape, D_OUT)
